# baseline (speedup 1.0000x reference)
import functools

import jax
import jax.numpy as jnp
from jax import lax
from jax.experimental import pallas as pl
from jax.experimental.pallas import tpu as pltpu

NBLK = 128
CH = 2560


def _body(wb_ref, nw_ref, idx_ref, msgA_ref, msgB_ref, rbf_ref,
          Wrbf_ref, Wup_ref, W1_ref, b1_ref, W2_ref, b2_ref, Wf_ref,
          out_ref, acc_ref, *, nch, nblocks):
    j = pl.program_id(0)
    chh = msgA_ref.shape[0]

    @pl.when(j == 0)
    def _():
        acc_ref[...] = jnp.zeros_like(acc_ref)

    @pl.when(j < nch)
    def _chunk():
        t = jnp.dot(rbf_ref[...], Wrbf_ref[...],
                    preferred_element_type=jnp.float32)
        ma = (msgA_ref[...] * t[0:chh]).astype(jnp.bfloat16)
        mb = (msgB_ref[...] * t[chh:]).astype(jnp.bfloat16)
        idx = idx_ref[0]
        niota = lax.broadcasted_iota(jnp.int32, (NBLK, 1), 0)
        base0 = wb_ref[j]

        def window(i, _):
            base = base0 + i * NBLK
            onehot = (idx - base == niota).astype(jnp.bfloat16)
            acc_ref[pl.ds(base, NBLK), :] += (
                jnp.dot(onehot[:, 0:chh], ma,
                        preferred_element_type=jnp.float32)
                + jnp.dot(onehot[:, chh:], mb,
                          preferred_element_type=jnp.float32))
            return 0

        lax.fori_loop(0, nw_ref[j], window, 0)

    @pl.when(j >= nch)
    def _mlp():
        g = j - nch
        a = acc_ref[pl.ds(g * NBLK, NBLK), :]
        h = jnp.dot(a, Wup_ref[...], preferred_element_type=jnp.float32)
        z = jnp.dot(h, W1_ref[...],
                    preferred_element_type=jnp.float32) + b1_ref[...]
        h = z * jax.nn.sigmoid(z)
        z = jnp.dot(h, W2_ref[...],
                    preferred_element_type=jnp.float32) + b2_ref[...]
        h = z * jax.nn.sigmoid(z)
        out_ref[...] = jnp.dot(h, Wf_ref[...],
                               preferred_element_type=jnp.float32)


def _run(messages, rbf, idx, W_rbf, W_up, W1, b1, W2, b2, W_final,
         n_particles, ch=CH, interpret=False):
    E, EMBED = messages.shape
    D_RBF = rbf.shape[1]
    OUT = W_up.shape[1]
    NT = W_final.shape[1]
    nblocks = (n_particles + NBLK - 1) // NBLK
    n_pad = nblocks * NBLK
    assert E % ch == 0
    nch = E // ch
    chh = ch // 2
    grid = nch + nblocks

    wbase = (idx[::ch] // 8) * 8
    last = idx[ch - 1::ch]
    nwin = (last - wbase) // NBLK + 1
    idx3 = idx.reshape(nch, 1, ch)

    grid_spec = pltpu.PrefetchScalarGridSpec(
        num_scalar_prefetch=2,
        grid=(grid,),
        in_specs=[
            pl.BlockSpec((1, 1, ch),
                         lambda j, wb, nw: (jnp.minimum(j, nch - 1), 0, 0)),
            pl.BlockSpec((chh, EMBED),
                         lambda j, wb, nw: (2 * jnp.minimum(j, nch - 1), 0)),
            pl.BlockSpec((chh, EMBED),
                         lambda j, wb, nw: (2 * jnp.minimum(j, nch - 1) + 1, 0)),
            pl.BlockSpec((ch, D_RBF),
                         lambda j, wb, nw: (jnp.minimum(j, nch - 1), 0)),
            pl.BlockSpec((D_RBF, EMBED), lambda j, wb, nw: (0, 0)),
            pl.BlockSpec((EMBED, OUT), lambda j, wb, nw: (0, 0)),
            pl.BlockSpec((OUT, OUT), lambda j, wb, nw: (0, 0)),
            pl.BlockSpec((1, OUT), lambda j, wb, nw: (0, 0)),
            pl.BlockSpec((OUT, OUT), lambda j, wb, nw: (0, 0)),
            pl.BlockSpec((1, OUT), lambda j, wb, nw: (0, 0)),
            pl.BlockSpec((OUT, NT), lambda j, wb, nw: (0, 0)),
        ],
        out_specs=pl.BlockSpec(
            (NBLK, NT), lambda j, wb, nw: (jnp.maximum(j - nch, 0), 0)),
        scratch_shapes=[
            pltpu.VMEM((n_pad + NBLK, EMBED), jnp.float32),
        ],
    )
    out = pl.pallas_call(
        functools.partial(_body, nch=nch, nblocks=nblocks),
        grid_spec=grid_spec,
        out_shape=jax.ShapeDtypeStruct((nblocks * NBLK, NT), jnp.float32),
        interpret=interpret,
    )(wbase.astype(jnp.int32), nwin.astype(jnp.int32),
      idx3, messages, messages, rbf, W_rbf, W_up, W1,
      b1.reshape(1, OUT), W2, b2.reshape(1, OUT), W_final)
    return out[:n_particles]


def kernel(messages, rbf, connectivity, W_rbf, W_up, W1, b1, W2, b2, W_final):
    idx = connectivity[0]
    return _run(messages, rbf, idx, W_rbf, W_up, W1, b1, W2, b2, W_final,
                n_particles=10000)


# speedup vs baseline: 1.0626x; 1.0626x over previous
import functools

import jax
import jax.numpy as jnp
from jax import lax
from jax.experimental import pallas as pl
from jax.experimental.pallas import tpu as pltpu

NBLK = 128
CH = 5000


def _body(wb_ref, nw_ref, idx_ref, msg_ref, rbf_ref,
          Wrbf_ref, Wup_ref, W1_ref, b1_ref, W2_ref, b2_ref, Wf_ref,
          out_ref, acc_ref, *, nch, nblocks):
    j = pl.program_id(0)

    @pl.when(j == 0)
    def _():
        acc_ref[...] = jnp.zeros_like(acc_ref)

    @pl.when(j < nch)
    def _chunk():
        t = jnp.dot(rbf_ref[...], Wrbf_ref[...],
                    preferred_element_type=jnp.float32)
        m = (msg_ref[...] * t).astype(jnp.bfloat16)
        idx = idx_ref[0]
        niota = lax.broadcasted_iota(jnp.int32, (NBLK, 1), 0)
        base0 = wb_ref[j]

        def window(i, _):
            base = base0 + i * NBLK
            onehot = (idx - base == niota).astype(jnp.bfloat16)
            acc_ref[pl.ds(base, NBLK), :] += jnp.dot(
                onehot, m, preferred_element_type=jnp.float32)
            return 0

        lax.fori_loop(0, nw_ref[j], window, 0)

    @pl.when(j >= nch)
    def _mlp():
        g = j - nch
        a = acc_ref[pl.ds(g * NBLK, NBLK), :]
        h = jnp.dot(a, Wup_ref[...], preferred_element_type=jnp.float32)
        z = jnp.dot(h, W1_ref[...],
                    preferred_element_type=jnp.float32) + b1_ref[...]
        h = z * jax.nn.sigmoid(z)
        z = jnp.dot(h, W2_ref[...],
                    preferred_element_type=jnp.float32) + b2_ref[...]
        h = z * jax.nn.sigmoid(z)
        out_ref[...] = jnp.dot(h, Wf_ref[...],
                               preferred_element_type=jnp.float32)


def _run(messages, rbf, idx, W_rbf, W_up, W1, b1, W2, b2, W_final,
         n_particles, ch=CH, interpret=False):
    E, EMBED = messages.shape
    D_RBF = rbf.shape[1]
    OUT = W_up.shape[1]
    NT = W_final.shape[1]
    nblocks = (n_particles + NBLK - 1) // NBLK
    n_pad = nblocks * NBLK
    assert E % ch == 0
    nch = E // ch
    grid = nch + nblocks

    wbase = (idx[::ch] // 8) * 8
    last = idx[ch - 1::ch]
    nwin = (last - wbase) // NBLK + 1
    idx3 = idx.reshape(nch, 1, ch)

    grid_spec = pltpu.PrefetchScalarGridSpec(
        num_scalar_prefetch=2,
        grid=(grid,),
        in_specs=[
            pl.BlockSpec((1, 1, ch),
                         lambda j, wb, nw: (jnp.minimum(j, nch - 1), 0, 0)),
            pl.BlockSpec((ch, EMBED),
                         lambda j, wb, nw: (jnp.minimum(j, nch - 1), 0)),
            pl.BlockSpec((ch, D_RBF),
                         lambda j, wb, nw: (jnp.minimum(j, nch - 1), 0)),
            pl.BlockSpec((D_RBF, EMBED), lambda j, wb, nw: (0, 0)),
            pl.BlockSpec((EMBED, OUT), lambda j, wb, nw: (0, 0)),
            pl.BlockSpec((OUT, OUT), lambda j, wb, nw: (0, 0)),
            pl.BlockSpec((1, OUT), lambda j, wb, nw: (0, 0)),
            pl.BlockSpec((OUT, OUT), lambda j, wb, nw: (0, 0)),
            pl.BlockSpec((1, OUT), lambda j, wb, nw: (0, 0)),
            pl.BlockSpec((OUT, NT), lambda j, wb, nw: (0, 0)),
        ],
        out_specs=pl.BlockSpec(
            (NBLK, NT), lambda j, wb, nw: (jnp.maximum(j - nch, 0), 0)),
        scratch_shapes=[
            pltpu.VMEM((n_pad + NBLK, EMBED), jnp.float32),
        ],
    )
    out = pl.pallas_call(
        functools.partial(_body, nch=nch, nblocks=nblocks),
        grid_spec=grid_spec,
        out_shape=jax.ShapeDtypeStruct((nblocks * NBLK, NT), jnp.float32),
        interpret=interpret,
    )(wbase.astype(jnp.int32), nwin.astype(jnp.int32),
      idx3, messages, rbf, W_rbf, W_up, W1,
      b1.reshape(1, OUT), W2, b2.reshape(1, OUT), W_final)
    return out[:n_particles]


def kernel(messages, rbf, connectivity, W_rbf, W_up, W1, b1, W2, b2, W_final):
    idx = connectivity[0]
    return _run(messages, rbf, idx, W_rbf, W_up, W1, b1, W2, b2, W_final,
                n_particles=10000)


# speedup vs baseline: 1.1390x; 1.0718x over previous
import functools

import jax
import jax.numpy as jnp
from jax import lax
from jax.experimental import pallas as pl
from jax.experimental.pallas import tpu as pltpu

NBLK = 128
CH = 8000


def _body(wb_ref, nw_ref, idx_ref, msg_ref, rbf_ref,
          Wrbf_ref, Wup_ref, W1_ref, b1_ref, W2_ref, b2_ref, Wf_ref,
          out_ref, acc_ref, *, nch, nblocks):
    j = pl.program_id(0)

    @pl.when(j == 0)
    def _():
        acc_ref[...] = jnp.zeros_like(acc_ref)

    @pl.when(j < nch)
    def _chunk():
        t = jnp.dot(rbf_ref[...], Wrbf_ref[...],
                    preferred_element_type=jnp.float32)
        m = (msg_ref[...] * t).astype(jnp.bfloat16)
        idx = idx_ref[0]
        niota = lax.broadcasted_iota(jnp.int32, (NBLK, 1), 0)
        base0 = wb_ref[j]

        def window(i, _):
            base = base0 + i * NBLK
            onehot = (idx - base == niota).astype(jnp.bfloat16)
            acc_ref[pl.ds(base, NBLK), :] += jnp.dot(
                onehot, m, preferred_element_type=jnp.float32)
            return 0

        lax.fori_loop(0, nw_ref[j], window, 0)

    @pl.when(j >= nch)
    def _mlp():
        g = j - nch
        a = acc_ref[pl.ds(g * NBLK, NBLK), :]
        h = jnp.dot(a, Wup_ref[...], preferred_element_type=jnp.float32)
        z = jnp.dot(h, W1_ref[...],
                    preferred_element_type=jnp.float32) + b1_ref[...]
        h = z * jax.nn.sigmoid(z)
        z = jnp.dot(h, W2_ref[...],
                    preferred_element_type=jnp.float32) + b2_ref[...]
        h = z * jax.nn.sigmoid(z)
        out_ref[...] = jnp.dot(h, Wf_ref[...],
                               preferred_element_type=jnp.float32)


def _run(messages, rbf, idx, W_rbf, W_up, W1, b1, W2, b2, W_final,
         n_particles, ch=CH, interpret=False):
    E, EMBED = messages.shape
    D_RBF = rbf.shape[1]
    OUT = W_up.shape[1]
    NT = W_final.shape[1]
    nblocks = (n_particles + NBLK - 1) // NBLK
    n_pad = nblocks * NBLK
    assert E % ch == 0
    nch = E // ch
    grid = nch + nblocks

    wbase = (idx[::ch] // 8) * 8
    last = idx[ch - 1::ch]
    nwin = (last - wbase) // NBLK + 1
    idx3 = idx.reshape(nch, 1, ch)

    grid_spec = pltpu.PrefetchScalarGridSpec(
        num_scalar_prefetch=2,
        grid=(grid,),
        in_specs=[
            pl.BlockSpec((1, 1, ch),
                         lambda j, wb, nw: (jnp.minimum(j, nch - 1), 0, 0)),
            pl.BlockSpec((ch, EMBED),
                         lambda j, wb, nw: (jnp.minimum(j, nch - 1), 0)),
            pl.BlockSpec((ch, D_RBF),
                         lambda j, wb, nw: (jnp.minimum(j, nch - 1), 0)),
            pl.BlockSpec((D_RBF, EMBED), lambda j, wb, nw: (0, 0)),
            pl.BlockSpec((EMBED, OUT), lambda j, wb, nw: (0, 0)),
            pl.BlockSpec((OUT, OUT), lambda j, wb, nw: (0, 0)),
            pl.BlockSpec((1, OUT), lambda j, wb, nw: (0, 0)),
            pl.BlockSpec((OUT, OUT), lambda j, wb, nw: (0, 0)),
            pl.BlockSpec((1, OUT), lambda j, wb, nw: (0, 0)),
            pl.BlockSpec((OUT, NT), lambda j, wb, nw: (0, 0)),
        ],
        out_specs=pl.BlockSpec(
            (NBLK, NT), lambda j, wb, nw: (jnp.maximum(j - nch, 0), 0)),
        scratch_shapes=[
            pltpu.VMEM((n_pad + NBLK, EMBED), jnp.float32),
        ],
    )
    out = pl.pallas_call(
        functools.partial(_body, nch=nch, nblocks=nblocks),
        grid_spec=grid_spec,
        out_shape=jax.ShapeDtypeStruct((nblocks * NBLK, NT), jnp.float32),
        interpret=interpret,
    )(wbase.astype(jnp.int32), nwin.astype(jnp.int32),
      idx3, messages, rbf, W_rbf, W_up, W1,
      b1.reshape(1, OUT), W2, b2.reshape(1, OUT), W_final)
    return out[:n_particles]


def kernel(messages, rbf, connectivity, W_rbf, W_up, W1, b1, W2, b2, W_final):
    idx = connectivity[0]
    return _run(messages, rbf, idx, W_rbf, W_up, W1, b1, W2, b2, W_final,
                n_particles=10000)
